# baseline (speedup 1.0000x reference)
import functools

import numpy as np
import jax
import jax.numpy as jnp
from jax import lax
from jax.experimental import pallas as pl
from jax.experimental.pallas import tpu as pltpu
from jax.experimental.pallas import tpu_sc as plsc

MAX_T = 1024
EMB_D = 256
NUM_FREQS = 10
OUT_D = EMB_D + 2 * NUM_FREQS
PAD_D = 280
BATCH = 16384
FRAMES = 16
ROW_D = FRAMES * OUT_D
CT = ROW_D // 8
BT = BATCH // 128
TPP = 69


def _table_body(emb_ref, out_ref):
    emb = emb_ref[:]
    t = lax.broadcasted_iota(jnp.int32, (MAX_T, NUM_FREQS), 0).astype(jnp.float32)
    j = lax.broadcasted_iota(jnp.int32, (MAX_T, NUM_FREQS), 1).astype(jnp.float32)
    raw = t * jnp.exp2(j) * np.float32(np.pi / MAX_T)
    pad = jnp.zeros((MAX_T, PAD_D - OUT_D), jnp.float32)
    out_ref[:] = jnp.concatenate([emb, jnp.sin(raw), jnp.cos(raw), pad], axis=1)


def _build_table(embed_table):
    return pl.pallas_call(
        _table_body,
        out_shape=jax.ShapeDtypeStruct((MAX_T, PAD_D), jnp.float32),
    )(embed_table)


def _sc_gather(idx_t, table):
    info = plsc.get_sparse_core_info()
    nc, ns, nl = info.num_cores, info.num_subcores, info.num_lanes
    nw = nc * ns
    bt_per_w = BT // nw
    n_pairs = FRAMES // 2
    n_vec = 18
    mesh = plsc.VectorSubcoreMesh(core_axis_name="c", subcore_axis_name="s")

    @functools.partial(
        pl.kernel,
        out_type=jax.ShapeDtypeStruct((CT, BT, 8, 128), jnp.float32),
        mesh=mesh,
        scratch_types=[
            pltpu.VMEM((FRAMES, 64), jnp.int32),
            pltpu.VMEM((2, 128, PAD_D), jnp.float32),
            pltpu.VMEM((35, 8, 64), jnp.float32),
            pltpu.VMEM((35, 8, 64), jnp.float32),
            pltpu.VMEM_SHARED((MAX_T, PAD_D), jnp.float32),
            pltpu.SemaphoreType.DMA,
            pltpu.SemaphoreType.DMA,
            pltpu.SemaphoreType.DMA,
            pltpu.SemaphoreType.DMA,
        ],
        compiler_params=pltpu.CompilerParams(use_tc_tiling_on_sc=False, needs_layout_passes=False),
    )
    def gather_kernel(
        idx_hbm, table_hbm, out_hbm, idx_v, rows_v, img0_v, img1_v, shared_v,
        g0, g1, w0, w1,
    ):
        img = (img0_v, img1_v)
        gsem = (g0, g1)
        wsem = (w0, w1)
        sid = lax.axis_index("s")
        wid = sid * nc + lax.axis_index("c")

        @pl.when(sid == 0)
        def _():
            pltpu.sync_copy(table_hbm, shared_v)

        plsc.subcore_barrier()

        iota = lax.broadcasted_iota(jnp.int32, (nl,), 0)

        def start_gathers(p, rb):
            for par in range(2):
                pltpu.async_copy(
                    shared_v.at[idx_v.at[2 * p + par]],
                    rows_v.at[rb, pl.ds(par * 64, 64)],
                    gsem[rb],
                )

        def wait_gathers(rb):
            for par in range(2):
                pltpu.make_async_copy(
                    table_hbm.at[pl.ds(0, 64)],
                    rows_v.at[rb, pl.ds(par * 64, 64)],
                    gsem[rb],
                ).wait()

        def start_write(bt, h, p, par):
            t0 = p * TPP
            if par == 0:
                pltpu.async_copy(
                    img0_v.at[pl.ds(0, 34)],
                    out_hbm.at[pl.ds(t0, 34), bt, :, pl.ds(h * 64, 64)],
                    wsem[0],
                )
                pltpu.async_copy(
                    img0_v.at[34, pl.ds(0, 4)],
                    out_hbm.at[t0 + 34, bt, pl.ds(0, 4), pl.ds(h * 64, 64)],
                    wsem[0],
                )
            else:
                pltpu.async_copy(
                    img1_v.at[0, pl.ds(4, 4)],
                    out_hbm.at[t0 + 34, bt, pl.ds(4, 4), pl.ds(h * 64, 64)],
                    wsem[1],
                )
                pltpu.async_copy(
                    img1_v.at[pl.ds(1, 34)],
                    out_hbm.at[pl.ds(t0 + 35, 34), bt, :, pl.ds(h * 64, 64)],
                    wsem[1],
                )

        def wait_write(par):
            if par == 0:
                pltpu.make_async_copy(
                    img0_v.at[pl.ds(0, 34)],
                    out_hbm.at[pl.ds(0, 34), 0, :, pl.ds(0, 64)],
                    wsem[0],
                ).wait()
                pltpu.make_async_copy(
                    img0_v.at[34, pl.ds(0, 4)],
                    out_hbm.at[34, 0, pl.ds(0, 4), pl.ds(0, 64)],
                    wsem[0],
                ).wait()
            else:
                pltpu.make_async_copy(
                    img1_v.at[0, pl.ds(4, 4)],
                    out_hbm.at[34, 0, pl.ds(4, 4), pl.ds(0, 64)],
                    wsem[1],
                ).wait()
                pltpu.make_async_copy(
                    img1_v.at[pl.ds(1, 34)],
                    out_hbm.at[pl.ds(1, 34), 0, :, pl.ds(0, 64)],
                    wsem[1],
                ).wait()

        def repack(rb, par):
            for v in range(n_vec):
                d0 = 16 * v if v < n_vec - 1 else OUT_D - nl
                row = iota + (d0 + 4 * par)
                i0 = row >> 3
                i1 = row & 7

                del i0, i1

                @plsc.parallel_loop(0, 64, unroll=2)
                def _(j):
                    x = rows_v[rb, par * 64 + j, pl.ds(d0, nl)]
                    row = lax.broadcasted_iota(jnp.int32, (nl,), 0) + (d0 + 4 * par)
                    i2 = jnp.zeros((nl,), jnp.int32) + j
                    plsc.store_scatter(img[par], [row >> 3, row & 7, i2], x)

        def block(bh, carry):
            bt4 = bh // 2
            h = bh % 2
            bt = wid * bt_per_w + bt4
            b0 = bt * 128 + h * 64
            pltpu.sync_copy(idx_hbm.at[:, pl.ds(b0, 64)], idx_v)
            start_gathers(0, 0)

            def pair2(p2, carry2):
                for ph in range(2):
                    p = p2 * 2 + ph
                    g = bh * 8 + p
                    wait_gathers(ph)

                    @pl.when(p + 1 < n_pairs)
                    def _():
                        start_gathers(p + 1, 1 - ph)

                    for par in range(2):
                        @pl.when(g >= 1)
                        def _():
                            wait_write(par)

                        repack(ph, par)
                        start_write(bt, h, p, par)
                return carry2

            lax.fori_loop(0, n_pairs // 2, pair2, 0)
            return carry

        lax.fori_loop(0, bt_per_w * 2, block, 0)
        wait_write(0)
        wait_write(1)

    return gather_kernel(idx_t, table)


def kernel(delta_t, embed_table):
    idx_t = jnp.transpose(delta_t).astype(jnp.int32)
    fused = _build_table(embed_table)
    p4 = _sc_gather(idx_t, fused)
    return p4.transpose(1, 3, 0, 2).reshape(BATCH, ROW_D)


# speedup vs baseline: 1.0003x; 1.0003x over previous
import functools

import numpy as np
import jax
import jax.numpy as jnp
from jax import lax
from jax.experimental import pallas as pl
from jax.experimental.pallas import tpu as pltpu
from jax.experimental.pallas import tpu_sc as plsc

MAX_T = 1024
EMB_D = 256
NUM_FREQS = 10
OUT_D = EMB_D + 2 * NUM_FREQS
PAD_D = 280
BATCH = 16384
FRAMES = 16
ROW_D = FRAMES * OUT_D
CT = ROW_D // 8
BT = BATCH // 128
TPP = 69


def _table_body(emb_ref, out_ref):
    emb = emb_ref[:]
    t = lax.broadcasted_iota(jnp.int32, (MAX_T, NUM_FREQS), 0).astype(jnp.float32)
    j = lax.broadcasted_iota(jnp.int32, (MAX_T, NUM_FREQS), 1).astype(jnp.float32)
    raw = t * jnp.exp2(j) * np.float32(np.pi / MAX_T)
    pad = jnp.zeros((MAX_T, PAD_D - OUT_D), jnp.float32)
    out_ref[:] = jnp.concatenate([emb, jnp.sin(raw), jnp.cos(raw), pad], axis=1)


def _build_table(embed_table):
    return pl.pallas_call(
        _table_body,
        out_shape=jax.ShapeDtypeStruct((MAX_T, PAD_D), jnp.float32),
    )(embed_table)


def _sc_gather(idx_t, table):
    info = plsc.get_sparse_core_info()
    nc, ns, nl = info.num_cores, info.num_subcores, info.num_lanes
    nw = nc * ns
    bt_per_w = BT // nw
    n_pairs = FRAMES // 2
    n_vec = 18
    mesh = plsc.VectorSubcoreMesh(core_axis_name="c", subcore_axis_name="s")

    @functools.partial(
        pl.kernel,
        out_type=jax.ShapeDtypeStruct((CT, BT, 8, 128), jnp.float32),
        mesh=mesh,
        scratch_types=[
            pltpu.VMEM((FRAMES, 64), jnp.int32),
            pltpu.VMEM((2, 128, PAD_D), jnp.float32),
            pltpu.VMEM((35, 8, 64), jnp.float32),
            pltpu.VMEM((35, 8, 64), jnp.float32),
            pltpu.VMEM_SHARED((MAX_T, PAD_D), jnp.float32),
            pltpu.SemaphoreType.DMA,
            pltpu.SemaphoreType.DMA,
            pltpu.SemaphoreType.DMA,
            pltpu.SemaphoreType.DMA,
        ],
        compiler_params=pltpu.CompilerParams(use_tc_tiling_on_sc=False, needs_layout_passes=False),
    )
    def gather_kernel(
        idx_hbm, table_hbm, out_hbm, idx_v, rows_v, img0_v, img1_v, shared_v,
        g0, g1, w0, w1,
    ):
        img = (img0_v, img1_v)
        gsem = (g0, g1)
        wsem = (w0, w1)
        sid = lax.axis_index("s")
        wid = sid * nc + lax.axis_index("c")

        @pl.when(sid == 0)
        def _():
            pltpu.sync_copy(table_hbm, shared_v)

        plsc.subcore_barrier()

        iota = lax.broadcasted_iota(jnp.int32, (nl,), 0)

        def start_gathers(p, rb):
            for par in range(2):
                pltpu.async_copy(
                    shared_v.at[idx_v.at[2 * p + par]],
                    rows_v.at[rb, pl.ds(par * 64, 64)],
                    gsem[rb],
                )

        def wait_gathers(rb):
            for par in range(2):
                pltpu.make_async_copy(
                    table_hbm.at[pl.ds(0, 64)],
                    rows_v.at[rb, pl.ds(par * 64, 64)],
                    gsem[rb],
                ).wait()

        def start_write(bt, h, p, par):
            t0 = p * TPP
            if par == 0:
                pltpu.async_copy(
                    img0_v.at[pl.ds(0, 34)],
                    out_hbm.at[pl.ds(t0, 34), bt, :, pl.ds(h * 64, 64)],
                    wsem[0],
                )
                pltpu.async_copy(
                    img0_v.at[34, pl.ds(0, 4)],
                    out_hbm.at[t0 + 34, bt, pl.ds(0, 4), pl.ds(h * 64, 64)],
                    wsem[0],
                )
            else:
                pltpu.async_copy(
                    img1_v.at[0, pl.ds(4, 4)],
                    out_hbm.at[t0 + 34, bt, pl.ds(4, 4), pl.ds(h * 64, 64)],
                    wsem[1],
                )
                pltpu.async_copy(
                    img1_v.at[pl.ds(1, 34)],
                    out_hbm.at[pl.ds(t0 + 35, 34), bt, :, pl.ds(h * 64, 64)],
                    wsem[1],
                )

        def wait_write(par):
            if par == 0:
                pltpu.make_async_copy(
                    img0_v.at[pl.ds(0, 34)],
                    out_hbm.at[pl.ds(0, 34), 0, :, pl.ds(0, 64)],
                    wsem[0],
                ).wait()
                pltpu.make_async_copy(
                    img0_v.at[34, pl.ds(0, 4)],
                    out_hbm.at[34, 0, pl.ds(0, 4), pl.ds(0, 64)],
                    wsem[0],
                ).wait()
            else:
                pltpu.make_async_copy(
                    img1_v.at[0, pl.ds(4, 4)],
                    out_hbm.at[34, 0, pl.ds(4, 4), pl.ds(0, 64)],
                    wsem[1],
                ).wait()
                pltpu.make_async_copy(
                    img1_v.at[pl.ds(1, 34)],
                    out_hbm.at[pl.ds(1, 34), 0, :, pl.ds(0, 64)],
                    wsem[1],
                ).wait()

        def repack(rb, par):
            for v in range(n_vec):
                d0 = 16 * v if v < n_vec - 1 else OUT_D - nl
                row = iota + (d0 + 4 * par)
                i0 = row >> 3
                i1 = row & 7
                zeros = jnp.zeros((nl,), jnp.int32)

                @plsc.parallel_loop(0, 64, unroll=2)
                def _(j):
                    x = rows_v[rb, par * 64 + j, pl.ds(d0, nl)]
                    plsc.store_scatter(img[par], [i0, i1, zeros + j], x)

        def block(bh, carry):
            bt4 = bh // 2
            h = bh % 2
            bt = wid * bt_per_w + bt4
            b0 = bt * 128 + h * 64
            pltpu.sync_copy(idx_hbm.at[:, pl.ds(b0, 64)], idx_v)
            start_gathers(0, 0)

            def pair2(p2, carry2):
                for ph in range(2):
                    p = p2 * 2 + ph
                    g = bh * 8 + p
                    wait_gathers(ph)

                    @pl.when(p + 1 < n_pairs)
                    def _():
                        start_gathers(p + 1, 1 - ph)

                    for par in range(2):
                        @pl.when(g >= 1)
                        def _():
                            wait_write(par)

                        repack(ph, par)
                        start_write(bt, h, p, par)
                return carry2

            lax.fori_loop(0, n_pairs // 2, pair2, 0)
            return carry

        lax.fori_loop(0, bt_per_w * 2, block, 0)
        wait_write(0)
        wait_write(1)

    return gather_kernel(idx_t, table)


def kernel(delta_t, embed_table):
    idx_t = jnp.transpose(delta_t).astype(jnp.int32)
    fused = _build_table(embed_table)
    p4 = _sc_gather(idx_t, fused)
    return p4.transpose(1, 3, 0, 2).reshape(BATCH, ROW_D)


# speedup vs baseline: 1.6402x; 1.6398x over previous
import functools

import numpy as np
import jax
import jax.numpy as jnp
from jax import lax
from jax.experimental import pallas as pl
from jax.experimental.pallas import tpu as pltpu
from jax.experimental.pallas import tpu_sc as plsc

MAX_T = 1024
EMB_D = 256
NUM_FREQS = 10
OUT_D = EMB_D + 2 * NUM_FREQS
PAD_D = 280


def _table_body(emb_ref, out_ref):
    emb = emb_ref[:]
    t = lax.broadcasted_iota(jnp.int32, (MAX_T, NUM_FREQS), 0).astype(jnp.float32)
    j = lax.broadcasted_iota(jnp.int32, (MAX_T, NUM_FREQS), 1).astype(jnp.float32)
    raw = t * jnp.exp2(j) * np.float32(np.pi / MAX_T)
    pad = jnp.zeros((MAX_T, PAD_D - OUT_D), jnp.float32)
    out_ref[:] = jnp.concatenate([emb, jnp.sin(raw), jnp.cos(raw), pad], axis=1)


def _build_table(embed_table):
    return pl.pallas_call(
        _table_body,
        out_shape=jax.ShapeDtypeStruct((MAX_T, PAD_D), jnp.float32),
    )(embed_table)


def _sc_gather(idx, table):
    n = idx.shape[0]
    info = plsc.get_sparse_core_info()
    nc, ns, nl = info.num_cores, info.num_subcores, info.num_lanes
    nw = nc * ns
    b_per_w = n // nw
    chunk = 64
    n_chunks = b_per_w // chunk
    n_vec = OUT_D // nl
    mesh = plsc.VectorSubcoreMesh(core_axis_name="c", subcore_axis_name="s")

    frames = 16
    rows_pc = chunk // frames
    row_d = frames * OUT_D

    @functools.partial(
        pl.kernel,
        out_type=jax.ShapeDtypeStruct((n // frames, row_d), jnp.float32),
        mesh=mesh,
        scratch_types=[
            pltpu.VMEM((b_per_w,), jnp.int32),
            pltpu.VMEM((2, chunk, PAD_D), jnp.float32),
            pltpu.VMEM((2, rows_pc, row_d), jnp.float32),
            pltpu.VMEM_SHARED((MAX_T, PAD_D), jnp.float32),
            pltpu.SemaphoreType.DMA,
            pltpu.SemaphoreType.DMA,
            pltpu.SemaphoreType.DMA,
            pltpu.SemaphoreType.DMA,
        ],
        compiler_params=pltpu.CompilerParams(use_tc_tiling_on_sc=False),
    )
    def gather_kernel(
        idx_hbm, table_hbm, out_hbm, idx_v, rows_v, flat_v, shared_v, g0, g1, w0, w1
    ):
        gsem = (g0, g1)
        wsem = (w0, w1)
        sid = lax.axis_index("s")
        wid = sid * nc + lax.axis_index("c")
        base = wid * b_per_w

        @pl.when(sid == 0)
        def _():
            pltpu.sync_copy(table_hbm, shared_v)

        plsc.subcore_barrier()
        pltpu.sync_copy(idx_hbm.at[pl.ds(base, b_per_w)], idx_v)

        def start_gather(k, b):
            pltpu.async_copy(
                shared_v.at[idx_v.at[pl.ds(k * chunk, chunk)]],
                rows_v.at[b],
                gsem[b],
            )

        def wait_gather(b):
            pltpu.make_async_copy(
                table_hbm.at[pl.ds(0, chunk)], rows_v.at[b], gsem[b]
            ).wait()

        def start_write(k, b):
            pltpu.async_copy(
                flat_v.at[b],
                out_hbm.at[pl.ds((base + k * chunk) // frames, rows_pc)],
                wsem[b],
            )

        def wait_write(b):
            pltpu.make_async_copy(
                flat_v.at[b], out_hbm.at[pl.ds(0, rows_pc)], wsem[b]
            ).wait()

        start_gather(0, 0)

        def outer(i, carry):
            k0 = i * 2
            for b in range(2):
                k = k0 + b

                @pl.when(k + 1 < n_chunks)
                def _():
                    start_gather(k + 1, 1 - b)

                wait_gather(b)

                @pl.when(k >= 2)
                def _():
                    wait_write(b)

                @plsc.parallel_loop(0, chunk, unroll=2)
                def row_body(r):
                    q = r // frames
                    dst = (r % frames) * OUT_D
                    for v in range(n_vec):
                        flat_v[b, q, pl.ds(dst + v * nl, nl)] = rows_v[
                            b, r, pl.ds(v * nl, nl)
                        ]
                    flat_v[b, q, pl.ds(dst + OUT_D - nl, nl)] = rows_v[
                        b, r, pl.ds(OUT_D - nl, nl)
                    ]
                start_write(k, b)
            return carry

        lax.fori_loop(0, n_chunks // 2, outer, 0)
        wait_write(0)
        wait_write(1)

    return gather_kernel(idx, table)


def kernel(delta_t, embed_table):
    idx = delta_t.reshape(-1).astype(jnp.int32)
    fused = _build_table(embed_table)
    return _sc_gather(idx, fused)


# speedup vs baseline: 1.6506x; 1.0063x over previous
import functools

import numpy as np
import jax
import jax.numpy as jnp
from jax import lax
from jax.experimental import pallas as pl
from jax.experimental.pallas import tpu as pltpu
from jax.experimental.pallas import tpu_sc as plsc

MAX_T = 1024
EMB_D = 256
NUM_FREQS = 10
OUT_D = EMB_D + 2 * NUM_FREQS
PAD_D = 280


def _table_body(emb_ref, out_ref):
    emb = emb_ref[:]
    t = lax.broadcasted_iota(jnp.int32, (MAX_T, NUM_FREQS), 0).astype(jnp.float32)
    j = lax.broadcasted_iota(jnp.int32, (MAX_T, NUM_FREQS), 1).astype(jnp.float32)
    raw = t * jnp.exp2(j) * np.float32(np.pi / MAX_T)
    pad = jnp.zeros((MAX_T, PAD_D - OUT_D), jnp.float32)
    out_ref[:] = jnp.concatenate([emb, jnp.sin(raw), jnp.cos(raw), pad], axis=1)


def _build_table(embed_table):
    return pl.pallas_call(
        _table_body,
        out_shape=jax.ShapeDtypeStruct((MAX_T, PAD_D), jnp.float32),
    )(embed_table)


def _sc_gather(idx, table):
    n = idx.shape[0]
    info = plsc.get_sparse_core_info()
    nc, ns, nl = info.num_cores, info.num_subcores, info.num_lanes
    nw = nc * ns
    b_per_w = n // nw
    chunk = 64
    n_chunks = b_per_w // chunk
    n_vec = OUT_D // nl
    mesh = plsc.VectorSubcoreMesh(core_axis_name="c", subcore_axis_name="s")

    frames = 16
    rows_pc = chunk // frames
    row_d = frames * OUT_D

    @functools.partial(
        pl.kernel,
        out_type=jax.ShapeDtypeStruct((n // frames, row_d), jnp.float32),
        mesh=mesh,
        scratch_types=[
            pltpu.VMEM((b_per_w,), jnp.int32),
            pltpu.VMEM((2, chunk, PAD_D), jnp.float32),
            pltpu.VMEM((2, rows_pc, row_d), jnp.float32),
            pltpu.VMEM_SHARED((MAX_T, PAD_D), jnp.float32),
            pltpu.SemaphoreType.DMA,
            pltpu.SemaphoreType.DMA,
            pltpu.SemaphoreType.DMA,
            pltpu.SemaphoreType.DMA,
        ],
        compiler_params=pltpu.CompilerParams(use_tc_tiling_on_sc=False),
    )
    def gather_kernel(
        idx_hbm, table_hbm, out_hbm, idx_v, rows_v, flat_v, shared_v, g0, g1, w0, w1
    ):
        gsem = (g0, g1)
        wsem = (w0, w1)
        sid = lax.axis_index("s")
        wid = sid * nc + lax.axis_index("c")
        base = wid * b_per_w

        @pl.when(sid == 0)
        def _():
            pltpu.sync_copy(table_hbm, shared_v)

        plsc.subcore_barrier()
        pltpu.sync_copy(idx_hbm.at[pl.ds(base, b_per_w)], idx_v)

        def start_gather(k, b):
            pltpu.async_copy(
                shared_v.at[idx_v.at[pl.ds(k * chunk, chunk)]],
                rows_v.at[b],
                gsem[b],
            )

        def wait_gather(b):
            pltpu.make_async_copy(
                table_hbm.at[pl.ds(0, chunk)], rows_v.at[b], gsem[b]
            ).wait()

        def start_write(k, b):
            pltpu.async_copy(
                flat_v.at[b],
                out_hbm.at[pl.ds((base + k * chunk) // frames, rows_pc)],
                wsem[b],
            )

        def wait_write(b):
            pltpu.make_async_copy(
                flat_v.at[b], out_hbm.at[pl.ds(0, rows_pc)], wsem[b]
            ).wait()

        start_gather(0, 0)

        def outer(i, carry):
            k0 = i * 2
            for b in range(2):
                k = k0 + b

                @pl.when(k + 1 < n_chunks)
                def _():
                    start_gather(k + 1, 1 - b)

                wait_gather(b)

                @pl.when(k >= 2)
                def _():
                    wait_write(b)

                @plsc.parallel_loop(0, chunk, unroll=4)
                def row_body(r):
                    q = r // frames
                    dst = (r % frames) * OUT_D
                    for v in range(n_vec):
                        flat_v[b, q, pl.ds(dst + v * nl, nl)] = rows_v[
                            b, r, pl.ds(v * nl, nl)
                        ]
                    flat_v[b, q, pl.ds(dst + OUT_D - nl, nl)] = rows_v[
                        b, r, pl.ds(OUT_D - nl, nl)
                    ]
                start_write(k, b)
            return carry

        lax.fori_loop(0, n_chunks // 2, outer, 0)
        wait_write(0)
        wait_write(1)

    return gather_kernel(idx, table)


def kernel(delta_t, embed_table):
    idx = delta_t.reshape(-1).astype(jnp.int32)
    fused = _build_table(embed_table)
    return _sc_gather(idx, fused)
